# baseline (speedup 1.0000x reference)
import jax
import jax.numpy as jnp
from jax.experimental import pallas as pl
from jax.experimental.pallas import tpu as pltpu

N = 100000
D = 128
H = 128
BLOCK_ROWS = 19200
VMEM_LIMIT = 100 * 1024 * 1024


def _body(src_ref, neigh_ref, ws_ref, wn_ref, out_ref):
    acc = jnp.dot(src_ref[...], ws_ref[...], preferred_element_type=jnp.float32)
    acc = acc + jnp.dot(neigh_ref[...], wn_ref[...],
                        preferred_element_type=jnp.float32)
    out_ref[...] = jnp.maximum(acc, 0.0)


def kernel(src_node_features, neighbor_node_features, W_self, W_neigh):
    grid = (pl.cdiv(N, BLOCK_ROWS),)
    return pl.pallas_call(
        _body,
        grid=grid,
        in_specs=[
            pl.BlockSpec((BLOCK_ROWS, D), lambda i: (i, 0)),
            pl.BlockSpec((BLOCK_ROWS, D), lambda i: (i, 0)),
            pl.BlockSpec((D, H), lambda i: (0, 0)),
            pl.BlockSpec((D, H), lambda i: (0, 0)),
        ],
        out_specs=pl.BlockSpec((BLOCK_ROWS, H), lambda i: (i, 0)),
        out_shape=jax.ShapeDtypeStruct((N, H), jnp.float32),
        compiler_params=pltpu.CompilerParams(
            dimension_semantics=("parallel",),
            vmem_limit_bytes=VMEM_LIMIT,
        ),
    )(src_node_features, neighbor_node_features, W_self, W_neigh)


# speedup vs baseline: 1.0357x; 1.0357x over previous
import jax
import jax.numpy as jnp
from jax.experimental import pallas as pl
from jax.experimental.pallas import tpu as pltpu

N = 100000
D = 128
H = 128
BLOCK_ROWS = 18000
VMEM_LIMIT = 100 * 1024 * 1024


def _body(src_ref, neigh_ref, ws_ref, wn_ref, out_ref):
    acc = jnp.dot(src_ref[...], ws_ref[...], preferred_element_type=jnp.float32)
    acc = acc + jnp.dot(neigh_ref[...], wn_ref[...],
                        preferred_element_type=jnp.float32)
    out_ref[...] = jnp.maximum(acc, 0.0)


def kernel(src_node_features, neighbor_node_features, W_self, W_neigh):
    grid = (pl.cdiv(N, BLOCK_ROWS),)
    return pl.pallas_call(
        _body,
        grid=grid,
        in_specs=[
            pl.BlockSpec((BLOCK_ROWS, D), lambda i: (i, 0)),
            pl.BlockSpec((BLOCK_ROWS, D), lambda i: (i, 0)),
            pl.BlockSpec((D, H), lambda i: (0, 0)),
            pl.BlockSpec((D, H), lambda i: (0, 0)),
        ],
        out_specs=pl.BlockSpec((BLOCK_ROWS, H), lambda i: (i, 0)),
        out_shape=jax.ShapeDtypeStruct((N, H), jnp.float32),
        compiler_params=pltpu.CompilerParams(
            dimension_semantics=("parallel",),
            vmem_limit_bytes=VMEM_LIMIT,
        ),
    )(src_node_features, neighbor_node_features, W_self, W_neigh)


# speedup vs baseline: 1.0379x; 1.0022x over previous
import jax
import jax.numpy as jnp
from jax.experimental import pallas as pl
from jax.experimental.pallas import tpu as pltpu

N = 100000
D = 128
H = 128
BLOCK_ROWS = 18000


def _body(src_ref, neigh_ref, ws_ref, wn_ref, out_ref):
    acc = jnp.dot(src_ref[...], ws_ref[...], preferred_element_type=jnp.float32)
    acc = acc + jnp.dot(neigh_ref[...], wn_ref[...],
                        preferred_element_type=jnp.float32)
    out_ref[...] = jnp.maximum(acc, 0.0)


def kernel(src_node_features, neighbor_node_features, W_self, W_neigh):
    grid = (pl.cdiv(N, BLOCK_ROWS),)
    return pl.pallas_call(
        _body,
        grid=grid,
        in_specs=[
            pl.BlockSpec((BLOCK_ROWS, D), lambda i: (i, 0)),
            pl.BlockSpec((BLOCK_ROWS, D), lambda i: (i, 0)),
            pl.BlockSpec((D, H), lambda i: (0, 0)),
            pl.BlockSpec((D, H), lambda i: (0, 0)),
        ],
        out_specs=pl.BlockSpec((BLOCK_ROWS, H), lambda i: (i, 0)),
        out_shape=jax.ShapeDtypeStruct((N, H), jnp.float32),
        compiler_params=pltpu.CompilerParams(
            dimension_semantics=("parallel",),
        ),
    )(src_node_features, neighbor_node_features, W_self, W_neigh)
